# baseline (speedup 1.0000x reference)
import functools

import jax
import jax.numpy as jnp
from jax import lax
from jax.experimental import pallas as pl
from jax.experimental.pallas import tpu as pltpu
from jax.experimental.pallas import tpu_sc as plsc

D = 32
MAX_NORM = 1.0
EPS = 1e-7

_NC = 2
_NS = 16
_NW = _NC * _NS
_L = 16
_NB = 3


def _make_kernel(Bq, Lq, R):
    B = Bq * Lq
    b_per_w = B // _NW
    q_per_w = Bq // _NW
    C = R * Lq
    nchunks = q_per_w // R
    ntrips = (nchunks - 2) // _NB
    assert Bq % _NW == 0 and q_per_w % R == 0 and C % _L == 0
    assert nchunks == 2 + _NB * ntrips and ntrips >= 1

    mesh = plsc.VectorSubcoreMesh(core_axis_name="c", subcore_axis_name="s")

    @functools.partial(
        pl.kernel,
        out_type=jax.ShapeDtypeStruct((Bq, Lq, D), jnp.float32),
        mesh=mesh,
        compiler_params=pltpu.CompilerParams(
            use_tc_tiling_on_sc=False, needs_layout_passes=False
        ),
        scratch_types=[
            pltpu.VMEM((b_per_w,), jnp.int32),
            pltpu.VMEM((C, D), jnp.float32),
            pltpu.VMEM((C, D), jnp.float32),
            pltpu.VMEM((C, D), jnp.float32),
            pltpu.VMEM((C,), jnp.float32),
            pltpu.SemaphoreType.DMA,
            pltpu.SemaphoreType.DMA,
            pltpu.SemaphoreType.DMA,
            pltpu.SemaphoreType.DMA,
            pltpu.SemaphoreType.DMA,
            pltpu.SemaphoreType.DMA,
        ],
    )
    def k(idx_hbm, table_hbm, out_hbm, idx_v, rows0, rows1, rows2, ssq_v,
          sem_g0, sem_g1, sem_g2, sem_o0, sem_o1, sem_o2):
        wid = lax.axis_index("s") * _NC + lax.axis_index("c")
        base = pl.multiple_of(wid * b_per_w, 8)
        qbase = wid * q_per_w
        pltpu.sync_copy(idx_hbm.at[pl.ds(base, b_per_w)], idx_v)

        rows = (rows0, rows1, rows2)
        sem_g = (sem_g0, sem_g1, sem_g2)
        sem_o = (sem_o0, sem_o1, sem_o2)
        iota16 = lax.iota(jnp.int32, 16)

        def start_gather(g, b):
            goff = pl.multiple_of(g * C, 8)
            pltpu.async_copy(
                table_hbm.at[idx_v.at[pl.ds(goff, C)]], rows[b], sem_g[b]
            )

        def wait_gather(b):
            pltpu.make_async_copy(
                table_hbm.at[pl.ds(0, C)], rows[b], sem_g[b]
            ).wait()

        def start_out(g, b):
            for i in range(R):
                pltpu.async_copy(
                    rows[b].at[pl.ds(i * Lq, Lq)],
                    out_hbm.at[qbase + g * R + i],
                    sem_o[b],
                )

        def wait_out(b):
            for i in range(R):
                pltpu.make_async_copy(
                    rows[b].at[pl.ds(i * Lq, Lq)],
                    out_hbm.at[qbase + i],
                    sem_o[b],
                ).wait()

        def process(g, b):
            rv = rows[b]

            @plsc.parallel_loop(
                0, C // _L, unroll=4,
                carry=jnp.zeros((_L,), jnp.float32),
            )
            def vmx(j, vmx):
                rowv = j * _L + iota16
                acc0 = jnp.zeros((_L,), jnp.float32)
                acc1 = jnp.zeros((_L,), jnp.float32)
                for c in range(0, D, 2):
                    c0 = jnp.full((_L,), c, jnp.int32)
                    c1 = jnp.full((_L,), c + 1, jnp.int32)
                    v0 = plsc.load_gather(rv, [rowv, c0])
                    v1 = plsc.load_gather(rv, [rowv, c1])
                    acc0 = acc0 + v0 * v0
                    acc1 = acc1 + v1 * v1
                acc = acc0 + acc1
                ssq_v[pl.ds(j * _L, _L)] = acc
                return jnp.maximum(vmx, acc)

            mx = jnp.max(vmx)

            @pl.when(mx > MAX_NORM * MAX_NORM)
            def _apply():
                def apply_body(j, c2):
                    rowv = j * _L + iota16
                    acc = ssq_v[pl.ds(j * _L, _L)]
                    xhalf = acc * 0.5
                    seed = 0x5F3759DF - (plsc.bitcast(acc, jnp.int32) >> 1)
                    y = plsc.bitcast(seed, jnp.float32)
                    y = y * (1.5 - xhalf * y * y)
                    y = y * (1.5 - xhalf * y * y)
                    y = y * (1.5 - xhalf * y * y)
                    norm = acc * y
                    scale = jnp.where(acc > MAX_NORM * MAX_NORM,
                                      1.0 / (norm + EPS), 1.0)
                    for c in range(D):
                        cv = jnp.full((_L,), c, jnp.int32)
                        v = plsc.load_gather(rv, [rowv, cv])
                        plsc.store_scatter(rv, [rowv, cv], v * scale)
                    return c2

                lax.fori_loop(0, C // _L, apply_body, 0)

            start_out(g, b)

        start_gather(0, 0)
        wait_gather(0)
        start_gather(1, 1)
        process(0, 0)
        wait_gather(1)
        start_gather(2, 2)
        process(1, 1)

        def trip_body(p, carry):
            g0 = _NB * p + 2
            for t in range(_NB):
                g = g0 + t
                b = (2 + t) % _NB
                nb = (b + 1) % _NB
                wait_gather(b)
                wait_out(nb)

                @pl.when(g + 1 < nchunks)
                def _():
                    start_gather(g + 1, nb)

                process(g, b)
            return carry

        lax.fori_loop(0, ntrips, trip_body, 0)
        wait_out(0)
        wait_out(1)

    return k


@jax.jit
def kernel(context, table):
    Bq, Lq = context.shape
    idx = context.reshape(Bq * Lq).astype(jnp.int32)
    return _make_kernel(Bq, Lq, 16)(idx, table)


# speedup vs baseline: 1.0196x; 1.0196x over previous
import functools

import jax
import jax.numpy as jnp
from jax import lax
from jax.experimental import pallas as pl
from jax.experimental.pallas import tpu as pltpu
from jax.experimental.pallas import tpu_sc as plsc

D = 32
MAX_NORM = 1.0
EPS = 1e-7

_NC = 2
_NS = 16
_NW = _NC * _NS
_L = 16
_NB = 3


def _make_kernel(Bq, Lq, R):
    B = Bq * Lq
    b_per_w = B // _NW
    q_per_w = Bq // _NW
    C = R * Lq
    nchunks = q_per_w // R
    ntrips = (nchunks - 2) // _NB
    assert Bq % _NW == 0 and q_per_w % R == 0 and C % _L == 0
    assert nchunks == 2 + _NB * ntrips and ntrips >= 1

    mesh = plsc.VectorSubcoreMesh(core_axis_name="c", subcore_axis_name="s")

    @functools.partial(
        pl.kernel,
        out_type=jax.ShapeDtypeStruct((Bq, Lq, D), jnp.float32),
        mesh=mesh,
        compiler_params=pltpu.CompilerParams(
            use_tc_tiling_on_sc=False, needs_layout_passes=False
        ),
        scratch_types=[
            pltpu.VMEM((b_per_w,), jnp.int32),
            pltpu.VMEM((C, D), jnp.float32),
            pltpu.VMEM((C, D), jnp.float32),
            pltpu.VMEM((C, D), jnp.float32),
            pltpu.VMEM((C,), jnp.float32),
            pltpu.SemaphoreType.DMA,
            pltpu.SemaphoreType.DMA,
            pltpu.SemaphoreType.DMA,
            pltpu.SemaphoreType.DMA,
            pltpu.SemaphoreType.DMA,
            pltpu.SemaphoreType.DMA,
        ],
    )
    def k(idx_hbm, table_hbm, out_hbm, idx_v, rows0, rows1, rows2, ssq_v,
          sem_g0, sem_g1, sem_g2, sem_o0, sem_o1, sem_o2):
        wid = lax.axis_index("s") * _NC + lax.axis_index("c")
        base = pl.multiple_of(wid * b_per_w, 8)
        qbase = wid * q_per_w
        pltpu.sync_copy(idx_hbm.at[pl.ds(base, b_per_w)], idx_v)

        rows = (rows0, rows1, rows2)
        sem_g = (sem_g0, sem_g1, sem_g2)
        sem_o = (sem_o0, sem_o1, sem_o2)
        iota16 = lax.iota(jnp.int32, 16)

        def start_gather(g, b):
            h = C // 2
            for s in range(2):
                goff = pl.multiple_of(g * C + s * h, 8)
                pltpu.async_copy(
                    table_hbm.at[idx_v.at[pl.ds(goff, h)]],
                    rows[b].at[pl.ds(s * h, h)],
                    sem_g[b],
                )

        def wait_gather(b):
            pltpu.make_async_copy(
                table_hbm.at[pl.ds(0, C)], rows[b], sem_g[b]
            ).wait()

        def start_out(g, b):
            for i in range(R):
                pltpu.async_copy(
                    rows[b].at[pl.ds(i * Lq, Lq)],
                    out_hbm.at[qbase + g * R + i],
                    sem_o[b],
                )

        def wait_out(b):
            for i in range(R):
                pltpu.make_async_copy(
                    rows[b].at[pl.ds(i * Lq, Lq)],
                    out_hbm.at[qbase + i],
                    sem_o[b],
                ).wait()

        def process(g, b):
            rv = rows[b]

            @plsc.parallel_loop(
                0, C // _L, unroll=2,
                carry=jnp.zeros((_L,), jnp.float32),
            )
            def vmx(j, vmx):
                rowv = j * _L + iota16
                acc0 = jnp.zeros((_L,), jnp.float32)
                acc1 = jnp.zeros((_L,), jnp.float32)
                for c in range(0, D, 2):
                    c0 = jnp.full((_L,), c, jnp.int32)
                    c1 = jnp.full((_L,), c + 1, jnp.int32)
                    v0 = plsc.load_gather(rv, [rowv, c0])
                    v1 = plsc.load_gather(rv, [rowv, c1])
                    acc0 = acc0 + v0 * v0
                    acc1 = acc1 + v1 * v1
                acc = acc0 + acc1
                ssq_v[pl.ds(j * _L, _L)] = acc
                return jnp.maximum(vmx, acc)

            mx = jnp.max(vmx)

            @pl.when(mx > MAX_NORM * MAX_NORM)
            def _apply():
                def apply_body(j, c2):
                    rowv = j * _L + iota16
                    acc = ssq_v[pl.ds(j * _L, _L)]
                    xhalf = acc * 0.5
                    seed = 0x5F3759DF - (plsc.bitcast(acc, jnp.int32) >> 1)
                    y = plsc.bitcast(seed, jnp.float32)
                    y = y * (1.5 - xhalf * y * y)
                    y = y * (1.5 - xhalf * y * y)
                    y = y * (1.5 - xhalf * y * y)
                    norm = acc * y
                    scale = jnp.where(acc > MAX_NORM * MAX_NORM,
                                      1.0 / (norm + EPS), 1.0)
                    for c in range(D):
                        cv = jnp.full((_L,), c, jnp.int32)
                        v = plsc.load_gather(rv, [rowv, cv])
                        plsc.store_scatter(rv, [rowv, cv], v * scale)
                    return c2

                lax.fori_loop(0, C // _L, apply_body, 0)

            start_out(g, b)

        start_gather(0, 0)
        wait_gather(0)
        start_gather(1, 1)
        process(0, 0)
        wait_gather(1)
        start_gather(2, 2)
        process(1, 1)

        def trip_body(p, carry):
            g0 = _NB * p + 2
            for t in range(_NB):
                g = g0 + t
                b = (2 + t) % _NB
                nb = (b + 1) % _NB
                wait_gather(b)
                wait_out(nb)

                @pl.when(g + 1 < nchunks)
                def _():
                    start_gather(g + 1, nb)

                process(g, b)
            return carry

        lax.fori_loop(0, ntrips, trip_body, 0)
        wait_out(0)
        wait_out(1)

    return k


@jax.jit
def kernel(context, table):
    Bq, Lq = context.shape
    idx = context.reshape(Bq * Lq).astype(jnp.int32)
    return _make_kernel(Bq, Lq, 16)(idx, table)


# speedup vs baseline: 1.0199x; 1.0004x over previous
import functools

import jax
import jax.numpy as jnp
from jax import lax
from jax.experimental import pallas as pl
from jax.experimental.pallas import tpu as pltpu
from jax.experimental.pallas import tpu_sc as plsc

D = 32
MAX_NORM = 1.0
EPS = 1e-7

_NC = 2
_NS = 16
_NW = _NC * _NS
_L = 16
_NB = 3


def _make_kernel(Bq, Lq, R):
    B = Bq * Lq
    b_per_w = B // _NW
    q_per_w = Bq // _NW
    C = R * Lq
    nchunks = q_per_w // R
    ntrips = (nchunks - 2) // _NB
    assert Bq % _NW == 0 and q_per_w % R == 0 and C % _L == 0
    assert nchunks == 2 + _NB * ntrips and ntrips >= 1

    mesh = plsc.VectorSubcoreMesh(core_axis_name="c", subcore_axis_name="s")

    @functools.partial(
        pl.kernel,
        out_type=jax.ShapeDtypeStruct((Bq, Lq, D), jnp.float32),
        mesh=mesh,
        compiler_params=pltpu.CompilerParams(
            use_tc_tiling_on_sc=False,
            needs_layout_passes=False,
            disable_bounds_checks=True,
        ),
        scratch_types=[
            pltpu.VMEM((b_per_w,), jnp.int32),
            pltpu.VMEM((C, D), jnp.float32),
            pltpu.VMEM((C, D), jnp.float32),
            pltpu.VMEM((C, D), jnp.float32),
            pltpu.VMEM((C,), jnp.float32),
            pltpu.SemaphoreType.DMA,
            pltpu.SemaphoreType.DMA,
            pltpu.SemaphoreType.DMA,
            pltpu.SemaphoreType.DMA,
            pltpu.SemaphoreType.DMA,
            pltpu.SemaphoreType.DMA,
        ],
    )
    def k(idx_hbm, table_hbm, out_hbm, idx_v, rows0, rows1, rows2, ssq_v,
          sem_g0, sem_g1, sem_g2, sem_o0, sem_o1, sem_o2):
        wid = lax.axis_index("s") * _NC + lax.axis_index("c")
        base = pl.multiple_of(wid * b_per_w, 8)
        qbase = wid * q_per_w
        pltpu.sync_copy(idx_hbm.at[pl.ds(base, b_per_w)], idx_v)

        rows = (rows0, rows1, rows2)
        sem_g = (sem_g0, sem_g1, sem_g2)
        sem_o = (sem_o0, sem_o1, sem_o2)
        iota16 = lax.iota(jnp.int32, 16)

        def start_gather(g, b):
            h = C // 2
            for s in range(2):
                goff = pl.multiple_of(g * C + s * h, 8)
                pltpu.async_copy(
                    table_hbm.at[idx_v.at[pl.ds(goff, h)]],
                    rows[b].at[pl.ds(s * h, h)],
                    sem_g[b],
                )

        def wait_gather(b):
            pltpu.make_async_copy(
                table_hbm.at[pl.ds(0, C)], rows[b], sem_g[b]
            ).wait()

        def start_out(g, b):
            for i in range(R):
                pltpu.async_copy(
                    rows[b].at[pl.ds(i * Lq, Lq)],
                    out_hbm.at[qbase + g * R + i],
                    sem_o[b],
                )

        def wait_out(b):
            for i in range(R):
                pltpu.make_async_copy(
                    rows[b].at[pl.ds(i * Lq, Lq)],
                    out_hbm.at[qbase + i],
                    sem_o[b],
                ).wait()

        def process(g, b):
            rv = rows[b]

            @plsc.parallel_loop(
                0, C // _L, unroll=2,
                carry=jnp.zeros((_L,), jnp.float32),
            )
            def vmx(j, vmx):
                rowv = j * _L + iota16
                acc0 = jnp.zeros((_L,), jnp.float32)
                acc1 = jnp.zeros((_L,), jnp.float32)
                for c in range(0, D, 2):
                    c0 = jnp.full((_L,), c, jnp.int32)
                    c1 = jnp.full((_L,), c + 1, jnp.int32)
                    v0 = plsc.load_gather(rv, [rowv, c0])
                    v1 = plsc.load_gather(rv, [rowv, c1])
                    acc0 = acc0 + v0 * v0
                    acc1 = acc1 + v1 * v1
                acc = acc0 + acc1
                ssq_v[pl.ds(j * _L, _L)] = acc
                return jnp.maximum(vmx, acc)

            mx = jnp.max(vmx)

            @pl.when(mx > MAX_NORM * MAX_NORM)
            def _apply():
                def apply_body(j, c2):
                    rowv = j * _L + iota16
                    acc = ssq_v[pl.ds(j * _L, _L)]
                    xhalf = acc * 0.5
                    seed = 0x5F3759DF - (plsc.bitcast(acc, jnp.int32) >> 1)
                    y = plsc.bitcast(seed, jnp.float32)
                    y = y * (1.5 - xhalf * y * y)
                    y = y * (1.5 - xhalf * y * y)
                    y = y * (1.5 - xhalf * y * y)
                    norm = acc * y
                    scale = jnp.where(acc > MAX_NORM * MAX_NORM,
                                      1.0 / (norm + EPS), 1.0)
                    for c in range(D):
                        cv = jnp.full((_L,), c, jnp.int32)
                        v = plsc.load_gather(rv, [rowv, cv])
                        plsc.store_scatter(rv, [rowv, cv], v * scale)
                    return c2

                lax.fori_loop(0, C // _L, apply_body, 0)

            start_out(g, b)

        start_gather(0, 0)
        wait_gather(0)
        start_gather(1, 1)
        process(0, 0)
        wait_gather(1)
        start_gather(2, 2)
        process(1, 1)

        def trip_body(p, carry):
            g0 = _NB * p + 2
            for t in range(_NB):
                g = g0 + t
                b = (2 + t) % _NB
                nb = (b + 1) % _NB
                wait_gather(b)
                wait_out(nb)

                @pl.when(g + 1 < nchunks)
                def _():
                    start_gather(g + 1, nb)

                process(g, b)
            return carry

        lax.fori_loop(0, ntrips, trip_body, 0)
        wait_out(0)
        wait_out(1)

    return k


@jax.jit
def kernel(context, table):
    Bq, Lq = context.shape
    idx = context.reshape(Bq * Lq).astype(jnp.int32)
    return _make_kernel(Bq, Lq, 16)(idx, table)


# speedup vs baseline: 1.2753x; 1.2504x over previous
import functools

import jax
import jax.numpy as jnp
from jax import lax
from jax.experimental import pallas as pl
from jax.experimental.pallas import tpu as pltpu
from jax.experimental.pallas import tpu_sc as plsc

D = 32
MAX_NORM = 1.0
EPS = 1e-7

_NC = 2
_NS = 16
_NW = _NC * _NS
_L = 16
_NB = 3


def _make_kernel(Bq, Lq, R):
    B = Bq * Lq
    b_per_w = B // _NW
    q_per_w = Bq // _NW
    C = R * Lq
    nchunks = q_per_w // R
    ntrips = (nchunks - 2) // _NB
    assert Bq % _NW == 0 and q_per_w % R == 0 and C % _L == 0
    assert nchunks == 2 + _NB * ntrips and ntrips >= 1

    mesh = plsc.VectorSubcoreMesh(core_axis_name="c", subcore_axis_name="s")

    @functools.partial(
        pl.kernel,
        out_type=jax.ShapeDtypeStruct((Bq, Lq, D), jnp.float32),
        mesh=mesh,
        compiler_params=pltpu.CompilerParams(
            use_tc_tiling_on_sc=False,
            needs_layout_passes=False,
            disable_bounds_checks=True,
        ),
        scratch_types=[
            pltpu.VMEM((b_per_w,), jnp.int32),
            pltpu.VMEM((C, D), jnp.float32),
            pltpu.VMEM((C, D), jnp.float32),
            pltpu.VMEM((C, D), jnp.float32),
            pltpu.SemaphoreType.DMA,
            pltpu.SemaphoreType.DMA,
            pltpu.SemaphoreType.DMA,
            pltpu.SemaphoreType.DMA,
            pltpu.SemaphoreType.DMA,
            pltpu.SemaphoreType.DMA,
        ],
    )
    def k(idx_hbm, table_hbm, out_hbm, idx_v, rows0, rows1, rows2,
          sem_g0, sem_g1, sem_g2, sem_o0, sem_o1, sem_o2):
        wid = lax.axis_index("s") * _NC + lax.axis_index("c")
        base = pl.multiple_of(wid * b_per_w, 8)
        qbase = wid * q_per_w
        pltpu.sync_copy(idx_hbm.at[pl.ds(base, b_per_w)], idx_v)

        rows = (rows0, rows1, rows2)
        sem_g = (sem_g0, sem_g1, sem_g2)
        sem_o = (sem_o0, sem_o1, sem_o2)
        iota16 = lax.iota(jnp.int32, 16)

        def start_gather(g, b):
            h = C // 2
            for s in range(2):
                goff = pl.multiple_of(g * C + s * h, 8)
                pltpu.async_copy(
                    table_hbm.at[idx_v.at[pl.ds(goff, h)]],
                    rows[b].at[pl.ds(s * h, h)],
                    sem_g[b],
                )

        def wait_gather(b):
            pltpu.make_async_copy(
                table_hbm.at[pl.ds(0, C)], rows[b], sem_g[b]
            ).wait()

        def start_out(g, b):
            for i in range(R):
                pltpu.async_copy(
                    rows[b].at[pl.ds(i * Lq, Lq)],
                    out_hbm.at[qbase + g * R + i],
                    sem_o[b],
                )

        def wait_out(b):
            for i in range(R):
                pltpu.make_async_copy(
                    rows[b].at[pl.ds(i * Lq, Lq)],
                    out_hbm.at[qbase + i],
                    sem_o[b],
                ).wait()

        def process(g, b):
            rv = rows[b]

            @plsc.parallel_loop(
                0, C // _L, unroll=2,
                carry=jnp.float32(0.0),
            )
            def mx(j, mx):
                base = j * _L
                for k in range(_L):
                    v0 = rv[base + k, pl.ds(0, _L)]
                    v1 = rv[base + k, pl.ds(_L, _L)]
                    s = jnp.sum(v0 * v0 + v1 * v1)
                    mx = jnp.maximum(mx, s)
                return mx

            @pl.when(mx > MAX_NORM * MAX_NORM)
            def _apply():
                def apply_body(j, c2):
                    rowv = j * _L + iota16
                    acc0 = jnp.zeros((_L,), jnp.float32)
                    acc1 = jnp.zeros((_L,), jnp.float32)
                    for c in range(0, D, 2):
                        c0 = jnp.full((_L,), c, jnp.int32)
                        c1 = jnp.full((_L,), c + 1, jnp.int32)
                        w0 = plsc.load_gather(rv, [rowv, c0])
                        w1 = plsc.load_gather(rv, [rowv, c1])
                        acc0 = acc0 + w0 * w0
                        acc1 = acc1 + w1 * w1
                    acc = acc0 + acc1
                    xhalf = acc * 0.5
                    seed = 0x5F3759DF - (plsc.bitcast(acc, jnp.int32) >> 1)
                    y = plsc.bitcast(seed, jnp.float32)
                    y = y * (1.5 - xhalf * y * y)
                    y = y * (1.5 - xhalf * y * y)
                    y = y * (1.5 - xhalf * y * y)
                    norm = acc * y
                    scale = jnp.where(acc > MAX_NORM * MAX_NORM,
                                      1.0 / (norm + EPS), 1.0)
                    for c in range(D):
                        cv = jnp.full((_L,), c, jnp.int32)
                        v = plsc.load_gather(rv, [rowv, cv])
                        plsc.store_scatter(rv, [rowv, cv], v * scale)
                    return c2

                lax.fori_loop(0, C // _L, apply_body, 0)

            start_out(g, b)

        start_gather(0, 0)
        wait_gather(0)
        start_gather(1, 1)
        process(0, 0)
        wait_gather(1)
        start_gather(2, 2)
        process(1, 1)

        def trip_body(p, carry):
            g0 = _NB * p + 2
            for t in range(_NB):
                g = g0 + t
                b = (2 + t) % _NB
                nb = (b + 1) % _NB
                wait_gather(b)
                wait_out(nb)

                @pl.when(g + 1 < nchunks)
                def _():
                    start_gather(g + 1, nb)

                process(g, b)
            return carry

        lax.fori_loop(0, ntrips, trip_body, 0)
        wait_out(0)
        wait_out(1)

    return k


@jax.jit
def kernel(context, table):
    Bq, Lq = context.shape
    idx = context.reshape(Bq * Lq).astype(jnp.int32)
    return _make_kernel(Bq, Lq, 16)(idx, table)
